# baseline (speedup 1.0000x reference)
import functools

import jax
import jax.numpy as jnp
from jax import lax
from jax.experimental import pallas as pl
from jax.experimental.pallas import tpu as pltpu

_TOPK = 64
_INT_MIN = -(2**31)
_INT_MAX = 2**31 - 1


def _sortable_key(z):
    zi = lax.bitcast_convert_type(z, jnp.int32)
    return jnp.where(zi < 0, zi ^ _INT_MAX, zi)


def _key_to_f32(s):
    fb = jnp.where(s < 0, s ^ _INT_MAX, s)
    return lax.bitcast_convert_type(fb, jnp.float32)


def _body(x_ref, we_ref, be_ref, wd_ref, bd_ref, recon_ref, zs_ref, s_ref,
          s16_ref, thr_ref, q_ref, *, nt, r, t, k, pos_bits):
    j = pl.program_id(1)

    @pl.when(j < nt)
    def _encode():
        z = jnp.dot(x_ref[...], we_ref[...],
                    preferred_element_type=jnp.float32) + be_ref[...]
        s = _sortable_key(z)
        s_ref[j] = s
        s16_ref[j] = lax.shift_right_arithmetic(s, 16).astype(jnp.int16)

    @pl.when(j == nt)
    def _select():
        def count_ge(c):
            acc = jnp.zeros((r, t), jnp.int32)
            for tile in range(nt):
                acc += (s_ref[tile] >= c).astype(jnp.int32)
            return jnp.sum(acc, axis=1, keepdims=True)

        def count_ge16(c16):
            acc = jnp.zeros((r, t), jnp.int16)
            c = c16.astype(jnp.int16)
            for tile in range(nt):
                acc += (s16_ref[tile] >= c).astype(jnp.int16)
            return jnp.sum(acc.astype(jnp.int32), axis=1, keepdims=True)

        def hi_step(it, thr16):
            cand = thr16 ^ lax.shift_left(jnp.int32(1), 15 - it)
            cand = lax.shift_right_arithmetic(lax.shift_left(cand, 16), 16)
            return jnp.where(count_ge16(cand) >= k, cand, thr16)

        thr16 = lax.fori_loop(0, 16, hi_step,
                              jnp.full((r, 1), -(2**15), jnp.int32))

        def lo_step(it, thr):
            cand = thr ^ lax.shift_left(jnp.int32(1), 15 - it)
            return jnp.where(count_ge(cand) >= k, cand, thr)

        thr = lax.fori_loop(0, 16, lo_step,
                            lax.shift_left(thr16, 16))
        thr_ref[...] = thr
        q_ref[...] = jnp.full((r, 1), nt * t, jnp.int32)

        c_ge = count_ge(thr)

        @pl.when(jnp.max(c_ge) > k)
        def _ties():
            quota = k - count_ge(thr + 1)

            def tie_cnt(qq):
                acc = jnp.zeros((r, t), jnp.int32)
                for tile in range(nt):
                    pos = lax.broadcasted_iota(jnp.int32, (r, t), 1) + tile * t
                    acc += ((s_ref[tile] == thr) & (pos <= qq)).astype(
                        jnp.int32)
                return jnp.sum(acc, axis=1, keepdims=True)

            def tie_step(it, qq):
                cand = qq + lax.shift_left(jnp.int32(1), pos_bits - 1 - it)
                return jnp.where(tie_cnt(cand) <= quota, cand, qq)

            q_ref[...] = lax.fori_loop(0, pos_bits, tie_step,
                                       jnp.full((r, 1), -1, jnp.int32))

    @pl.when(j >= nt)
    def _decode():
        jj = j - nt
        s = s_ref[jj]
        thr = thr_ref[...]
        pos = lax.broadcasted_iota(jnp.int32, (r, t), 1) + jj * t
        keep = (s > thr) | ((s == thr) & (pos <= q_ref[...]))
        zst = jnp.where(keep, _key_to_f32(s), 0.0)
        zs_ref[...] = zst
        acc = jnp.dot(zst.astype(jnp.bfloat16), wd_ref[...],
                      preferred_element_type=jnp.float32)

        @pl.when(jj == 0)
        def _():
            recon_ref[...] = acc + bd_ref[...]

        @pl.when(jj > 0)
        def _():
            recon_ref[...] += acc


@jax.jit
def kernel(x, W_enc, b_enc, W_dec, b_dec):
    n, d = x.shape
    h = W_enc.shape[1]
    t = min(1024, h)
    nt = h // t
    r = min(512, n)
    ni = n // r
    pos_bits = max(1, (h + 1).bit_length())

    body = functools.partial(_body, nt=nt, r=r, t=t, k=_TOPK,
                             pos_bits=pos_bits)

    recon, z_sparse = pl.pallas_call(
        body,
        grid=(ni, 2 * nt),
        in_specs=[
            pl.BlockSpec((r, d), lambda i, j: (i, 0)),
            pl.BlockSpec((d, t), lambda i, j: (0, jnp.minimum(j, nt - 1))),
            pl.BlockSpec((1, t), lambda i, j: (0, jnp.minimum(j, nt - 1))),
            pl.BlockSpec((t, d), lambda i, j: (jnp.maximum(j - nt, 0), 0)),
            pl.BlockSpec((1, d), lambda i, j: (0, 0)),
        ],
        out_specs=[
            pl.BlockSpec((r, d), lambda i, j: (i, 0)),
            pl.BlockSpec((r, t), lambda i, j: (i, jnp.maximum(j - nt, 0))),
        ],
        out_shape=[
            jax.ShapeDtypeStruct((n, d), jnp.float32),
            jax.ShapeDtypeStruct((n, h), jnp.float32),
        ],
        scratch_shapes=[
            pltpu.VMEM((nt, r, t), jnp.int32),
            pltpu.VMEM((nt, r, t), jnp.int16),
            pltpu.VMEM((r, 1), jnp.int32),
            pltpu.VMEM((r, 1), jnp.int32),
        ],
        compiler_params=pltpu.CompilerParams(
            dimension_semantics=("arbitrary", "arbitrary")),
    )(x, W_enc, b_enc.reshape(1, h), W_dec.astype(jnp.bfloat16),
      b_dec.reshape(1, d))
    return (recon, z_sparse)
